# Initial kernel scaffold; baseline (speedup 1.0000x reference)
import jax
import jax.numpy as jnp
from jax.experimental import pallas as pl


def kernel(x, pos, params, edge_index):
    raise NotImplementedError("write your pallas kernel here")


# speedup vs baseline: 1.9867x; 1.9867x over previous
import jax
import jax.numpy as jnp
from jax import lax
from jax.experimental import pallas as pl
from jax.experimental.pallas import tpu as pltpu
from jax.experimental.pallas import tpu_sc as plsc

_N = 10000
_E = 320000
_DH = 128
_H = 512
_PP = 16
_TW = 256

_KE = 512
_KN = 1000
_KP = 400

_NW = 32
_GB = 80
_SB = 80
_NPAD = 10240
_NSTRIPE = _NPAD // 16


def _silu(v):
    return v * jax.lax.logistic(v)


def _sc_gather_body(tbl, src, dst, gs, gd, sidx, didx, bs, bd, sem_s, sem_d):
    wid = lax.axis_index("s") * 2 + lax.axis_index("c")
    per_w = _E // _NW
    base = wid * per_w

    def chunk(i, carry):
        e0 = base + i * _GB
        pltpu.sync_copy(src.at[pl.ds(e0, _GB)], sidx)
        pltpu.sync_copy(dst.at[pl.ds(e0, _GB)], didx)
        cs = pltpu.async_copy(tbl.at[sidx], bs, sem_s)
        cd = pltpu.async_copy(tbl.at[didx], bd, sem_d)
        cs.wait()
        cd.wait()
        pltpu.sync_copy(bs, gs.at[pl.ds(e0, _GB)])
        pltpu.sync_copy(bd, gd.at[pl.ds(e0, _GB)])
        return carry

    lax.fori_loop(0, per_w // _GB, chunk, 0)


def _sc_gather(tbl, src, dst):
    mesh = plsc.VectorSubcoreMesh(core_axis_name="c", subcore_axis_name="s")
    f = pl.kernel(
        _sc_gather_body,
        mesh=mesh,
        out_type=[
            jax.ShapeDtypeStruct((_E, _TW), jnp.float32),
            jax.ShapeDtypeStruct((_E, _TW), jnp.float32),
        ],
        scratch_types=[
            pltpu.VMEM((_GB,), jnp.int32),
            pltpu.VMEM((_GB,), jnp.int32),
            pltpu.VMEM((_GB, _TW), jnp.float32),
            pltpu.VMEM((_GB, _TW), jnp.float32),
            pltpu.SemaphoreType.DMA,
            pltpu.SemaphoreType.DMA,
        ],
    )
    return f(tbl, src, dst)


def _sc_scatter4_body(m0, m1, m2, m3, mx, dst, z128,
                      hn0, hn1, hn2, hn3, xs0, xs1, dbuf, mbuf, acc):
    core = lax.axis_index("c")
    sid = lax.axis_index("s")
    r0 = sid * _NSTRIPE
    per_tile = _E // 16
    nchunks = per_tile // _SB

    def pass_m(mref, out):
        pltpu.sync_copy(z128, acc.at[pl.ds(r0, _NSTRIPE)])
        plsc.subcore_barrier()

        def chunk(i, carry):
            e0 = sid * per_tile + i * _SB
            pltpu.sync_copy(dst.at[pl.ds(e0, _SB)], dbuf)
            pltpu.sync_copy(mref.at[pl.ds(e0, _SB)], mbuf)
            pltpu.sync_copy(mbuf, acc.at[dbuf], add=True)
            return carry

        lax.fori_loop(0, nchunks, chunk, 0)
        plsc.subcore_barrier()
        pltpu.sync_copy(acc.at[pl.ds(r0, _NSTRIPE)],
                        out.at[pl.ds(r0, _NSTRIPE)])

    def pass_half(mref, out):
        pltpu.sync_copy(z128, acc.at[pl.ds(r0, _NSTRIPE)])
        plsc.subcore_barrier()
        half = _E // 2
        per_t = half // 16

        def chunk(i, carry):
            e0 = core * half + sid * per_t + i * _SB
            pltpu.sync_copy(dst.at[pl.ds(e0, _SB)], dbuf)
            pltpu.sync_copy(mref.at[pl.ds(e0, _SB)], mbuf)
            pltpu.sync_copy(mbuf, acc.at[dbuf], add=True)
            return carry

        lax.fori_loop(0, per_t // _SB, chunk, 0)
        plsc.subcore_barrier()
        pltpu.sync_copy(acc.at[pl.ds(r0, _NSTRIPE)],
                        out.at[pl.ds(r0, _NSTRIPE)])

    @pl.when(core == 0)
    def _():
        pass_m(m0, hn0)
        pass_m(m1, hn1)
        pass_half(mx, xs0)

    @pl.when(core == 1)
    def _():
        pass_m(m2, hn2)
        pass_m(m3, hn3)
        pass_half(mx, xs1)


def _sc_scatter4(m0, m1, m2, m3, mx, dst, z128):
    mesh = plsc.VectorSubcoreMesh(core_axis_name="c", subcore_axis_name="s")
    f = pl.kernel(
        _sc_scatter4_body,
        mesh=mesh,
        out_type=[jax.ShapeDtypeStruct((_NPAD, _DH), jnp.float32)] * 6,
        scratch_types=[
            pltpu.VMEM((_SB,), jnp.int32),
            pltpu.VMEM((_SB, _DH), jnp.float32),
            pltpu.VMEM_SHARED((_NPAD, _DH), jnp.float32),
        ],
    )
    return f(m0, m1, m2, m3, mx, dst, z128)


def _sc_scatter_body(m0, m1, m2, m3, mx, dst, z128, z16,
                     hn0, hn1, hn2, hn3, xs,
                     dbuf, mbuf, xbuf, acc, accx):
    core = lax.axis_index("c")
    sid = lax.axis_index("s")
    r0 = sid * _NSTRIPE
    per_tile = _E // 16
    nchunks = per_tile // _SB
    ms = (m0, m1, m2, m3)
    hns = (hn0, hn1, hn2, hn3)

    def scatter_pass(mref):
        def chunk(i, carry):
            e0 = sid * per_tile + i * _SB
            pltpu.sync_copy(dst.at[pl.ds(e0, _SB)], dbuf)
            pltpu.sync_copy(mref.at[pl.ds(e0, _SB)], mbuf)
            pltpu.sync_copy(mbuf, acc.at[dbuf], add=True)
            return carry

        lax.fori_loop(0, nchunks, chunk, 0)

    for q in range(2):
        pltpu.sync_copy(z128, acc.at[pl.ds(r0, _NSTRIPE)])
        plsc.subcore_barrier()

        @pl.when(core == 0)
        def _(q=q):
            scatter_pass(ms[q])

        @pl.when(core == 1)
        def _(q=q):
            scatter_pass(ms[2 + q])

        plsc.subcore_barrier()

        @pl.when(core == 0)
        def _(q=q):
            pltpu.sync_copy(acc.at[pl.ds(r0, _NSTRIPE)],
                            hns[q].at[pl.ds(r0, _NSTRIPE)])

        @pl.when(core == 1)
        def _(q=q):
            pltpu.sync_copy(acc.at[pl.ds(r0, _NSTRIPE)],
                            hns[2 + q].at[pl.ds(r0, _NSTRIPE)])

        plsc.subcore_barrier()

    pltpu.sync_copy(z16, accx.at[pl.ds(r0, _NSTRIPE)])
    plsc.subcore_barrier()

    @pl.when(core == 0)
    def _():
        def chunkx(i, carry):
            e0 = sid * per_tile + i * _SB
            pltpu.sync_copy(dst.at[pl.ds(e0, _SB)], dbuf)
            pltpu.sync_copy(mx.at[pl.ds(e0, _SB)], xbuf)
            pltpu.sync_copy(xbuf, accx.at[dbuf], add=True)
            return carry

        lax.fori_loop(0, nchunks, chunkx, 0)

    plsc.subcore_barrier()

    @pl.when(core == 0)
    def _():
        pltpu.sync_copy(accx.at[pl.ds(r0, _NSTRIPE)],
                        xs.at[pl.ds(r0, _NSTRIPE)])


def _sc_scatter(m0, m1, m2, m3, mx, dst, z128, z16):
    mesh = plsc.VectorSubcoreMesh(core_axis_name="c", subcore_axis_name="s")
    f = pl.kernel(
        _sc_scatter_body,
        mesh=mesh,
        out_type=[
            jax.ShapeDtypeStruct((_NPAD, _DH), jnp.float32),
            jax.ShapeDtypeStruct((_NPAD, _DH), jnp.float32),
            jax.ShapeDtypeStruct((_NPAD, _DH), jnp.float32),
            jax.ShapeDtypeStruct((_NPAD, _DH), jnp.float32),
            jax.ShapeDtypeStruct((_NPAD, _PP), jnp.float32),
        ],
        scratch_types=[
            pltpu.VMEM((_SB,), jnp.int32),
            pltpu.VMEM((_SB, _DH), jnp.float32),
            pltpu.VMEM((_SB, _PP), jnp.float32),
            pltpu.VMEM_SHARED((_NPAD, _DH), jnp.float32),
            pltpu.VMEM_SHARED((_NPAD, _PP), jnp.float32),
        ],
    )
    return f(m0, m1, m2, m3, mx, dst, z128, z16)


def _tc_edge_body(gs, gd, Ws, Wd, wr, be1, We2, be2, Wc1, bc1, wc2,
                  m0, m1, m2, m3, mxo):
    hs = gs[:, :_DH]
    ps = gs[:, _DH:_DH + _PP]
    hd = gd[:, :_DH]
    pd = gd[:, _DH:_DH + _PP]
    xdr = ps - pd
    radial = jnp.sum(xdr * xdr, axis=1, keepdims=True)
    z1 = (jnp.dot(hs, Ws[...], preferred_element_type=jnp.float32)
          + jnp.dot(hd, Wd[...], preferred_element_type=jnp.float32)
          + radial * wr[...] + be1[...])
    a1 = _silu(z1)
    m = _silu(jnp.dot(a1, We2[...], preferred_element_type=jnp.float32)
              + be2[...])
    c1 = _silu(jnp.dot(m, Wc1[...], preferred_element_type=jnp.float32)
               + bc1[...])
    c = jnp.sum(c1 * wc2[...], axis=1, keepdims=True)
    inv = 1.0 / (jnp.sqrt(radial) + 1e-30)
    mx = c * (xdr * inv)
    mx = jnp.concatenate(
        [mx, jnp.zeros((mx.shape[0], _DH - _PP), jnp.float32)], axis=1)
    lane = lax.broadcasted_iota(jnp.int32, mx.shape, 1)
    mx = jnp.where(lane == 3, 1.0, mx)
    m0[...] = m[:, 0 * _DH:1 * _DH]
    m1[...] = m[:, 1 * _DH:2 * _DH]
    m2[...] = m[:, 2 * _DH:3 * _DH]
    m3[...] = m[:, 3 * _DH:4 * _DH]
    mxo[...] = mx


def _tc_edge(gs, gd, Ws, Wd, wr, be1, We2, be2, Wc1, bc1, wc2):
    grid = (_E // _KE,)
    ew = pl.BlockSpec((_KE, _TW), lambda i: (i, 0))
    full = lambda a, b: pl.BlockSpec((a, b), lambda i: (0, 0))
    mo = pl.BlockSpec((_KE, _DH), lambda i: (i, 0))
    return pl.pallas_call(
        _tc_edge_body,
        grid=grid,
        in_specs=[ew, ew,
                  full(_DH, _H), full(_DH, _H), full(1, _H), full(1, _H),
                  full(_H, _H), full(1, _H), full(_H, _H), full(1, _H),
                  full(1, _H)],
        out_specs=[mo, mo, mo, mo, mo],
        out_shape=[jax.ShapeDtypeStruct((_E, _DH), jnp.float32)] * 5,
        compiler_params=pltpu.CompilerParams(
            dimension_semantics=("arbitrary",)),
    )(gs, gd, Ws, Wd, wr, be1, We2, be2, Wc1, bc1, wc2)


def _tc_node_body(h, hn0, hn1, hn2, hn3, xp, xs0, xs1, W1h, W1n, bn1, Wn2,
                  bn2, hnew, xnew, tbl):
    hne = jnp.concatenate([hn0[...], hn1[...], hn2[...], hn3[...]], axis=1)
    s = (xs0[...] + xs1[...])[:, :_PP]
    cnt = jnp.maximum(s[:, 3:4], 1.0)
    xv = xp[...] + s / cnt
    lane = lax.broadcasted_iota(jnp.int32, xv.shape, 1)
    xv = jnp.where(lane < 3, xv, 0.0)
    hh = _silu(jnp.dot(h[...], W1h[...], preferred_element_type=jnp.float32)
               + jnp.dot(hne, W1n[...], preferred_element_type=jnp.float32)
               + bn1[...])
    hn = jnp.dot(hh, Wn2[...], preferred_element_type=jnp.float32) + bn2[...]
    hnew[...] = hn
    xnew[...] = xv
    tbl[...] = jnp.concatenate(
        [hn, xv, jnp.zeros((hn.shape[0], _TW - _DH - _PP), jnp.float32)],
        axis=1)


def _tc_node(h, hn0, hn1, hn2, hn3, xp, xs0, xs1, W1h, W1n, bn1, Wn2, bn2):
    grid = (_N // _KN,)
    nb = lambda w: pl.BlockSpec((_KN, w), lambda i: (i, 0))
    full = lambda a, b: pl.BlockSpec((a, b), lambda i: (0, 0))
    return pl.pallas_call(
        _tc_node_body,
        grid=grid,
        in_specs=[nb(_DH), nb(_DH), nb(_DH), nb(_DH), nb(_DH),
                  nb(_PP), nb(_DH), nb(_DH),
                  full(_DH, _H), full(_H, _H), full(1, _H),
                  full(_H, _DH), full(1, _DH)],
        out_specs=[nb(_DH), nb(_PP), nb(_TW)],
        out_shape=[jax.ShapeDtypeStruct((_N, _DH), jnp.float32),
                   jax.ShapeDtypeStruct((_N, _PP), jnp.float32),
                   jax.ShapeDtypeStruct((_N, _TW), jnp.float32)],
        compiler_params=pltpu.CompilerParams(
            dimension_semantics=("arbitrary",)),
    )(h, hn0, hn1, hn2, hn3, xp, xs0, xs1, W1h, W1n, bn1, Wn2, bn2)


def _tc_pool_body(hf, xf, out):
    g = _KP // 5
    r = lax.broadcasted_iota(jnp.int32, (g, _KP), 0)
    c = lax.broadcasted_iota(jnp.int32, (g, _KP), 1)
    P = jnp.where(r == c // 5, 0.2, 0.0).astype(jnp.float32)
    hm = jnp.dot(P, hf[...], preferred_element_type=jnp.float32)
    xm = jnp.dot(P, xf[...], preferred_element_type=jnp.float32)
    out[...] = jnp.concatenate([hm, xm[:, :3]], axis=1)


def _tc_pool(hf, xf):
    grid = (_N // _KP,)
    g = _KP // 5
    return pl.pallas_call(
        _tc_pool_body,
        grid=grid,
        in_specs=[pl.BlockSpec((_KP, _DH), lambda i: (i, 0)),
                  pl.BlockSpec((_KP, _PP), lambda i: (i, 0))],
        out_specs=pl.BlockSpec((g, _DH + 3), lambda i: (i, 0)),
        out_shape=jax.ShapeDtypeStruct((_N // 5, _DH + 3), jnp.float32),
        compiler_params=pltpu.CompilerParams(
            dimension_semantics=("arbitrary",)),
    )(hf, xf)


def kernel(x, pos, params, edge_index):
    src = edge_index[0]
    dst = edge_index[1]
    xp = jnp.pad(pos, ((0, 0), (0, _PP - 3)))
    tbl = jnp.concatenate(
        [x, xp, jnp.zeros((x.shape[0], _TW - _DH - _PP), jnp.float32)],
        axis=1)
    z128 = jnp.zeros((_NSTRIPE, _DH), jnp.float32)
    h = x
    for p in params:
        Ws = p['We1'][:_DH]
        Wd = p['We1'][_DH:2 * _DH]
        wr = p['We1'][2 * _DH:]
        wc2 = p['Wc2'].T
        W1h = p['Wn1'][:_DH]
        W1n = p['Wn1'][_DH:]
        gs, gd = _sc_gather(tbl, src, dst)
        m0, m1, m2, m3, mx = _tc_edge(
            gs, gd, Ws, Wd, wr, p['be1'][None], p['We2'], p['be2'][None],
            p['Wc1'], p['bc1'][None], wc2)
        hn0, hn1, hn2, hn3, xs0, xs1 = _sc_scatter4(m0, m1, m2, m3, mx,
                                                    dst, z128)
        h, xp, tbl = _tc_node(h, hn0, hn1, hn2, hn3, xp, xs0, xs1,
                              W1h, W1n, p['bn1'][None], p['Wn2'],
                              p['bn2'][None])
    return _tc_pool(h, xp)
